# baseline (speedup 1.0000x reference)
import functools

import jax
import jax.numpy as jnp
import numpy as np
from jax import lax
from jax.experimental import pallas as pl
from jax.experimental.pallas import tpu as pltpu
from jax.experimental.pallas import tpu_sc as plsc

_GROUP_SIZE = 4
_NC = 2
_NS = 16

_BATCH, _STREAMS, _CH, _T = 32, 4, 2, 44100
_ROWS = _BATCH * _STREAMS
_CW = 11008
_CWT = _T - 3 * _CW
_CHUNKS = ((0, _CW), (_CW, _CW), (2 * _CW, _CW), (3 * _CW, _CWT))
_NBUF = 4

_PERM = (
    (1, 1, 1, 0, 0, 0, 2, 1, 2, 2, 3, 3, 3, 3, 0, 2),
    (1, 1, 1, 2, 2, 0, 0, 3, 3, 2, 2, 1, 0, 3, 3, 0),
    (0, 3, 1, 1, 2, 1, 2, 0, 3, 2, 0, 2, 1, 0, 3, 3),
    (3, 3, 0, 3, 2, 0, 2, 0, 1, 2, 1, 1, 0, 1, 3, 2),
    (0, 0, 2, 0, 3, 3, 3, 1, 1, 2, 1, 3, 2, 1, 0, 2),
    (1, 1, 1, 2, 3, 3, 3, 0, 0, 2, 2, 1, 2, 0, 0, 3),
    (3, 3, 1, 1, 0, 0, 2, 2, 1, 1, 0, 0, 2, 2, 3, 3),
    (0, 3, 2, 0, 2, 2, 0, 1, 1, 1, 3, 2, 3, 0, 1, 3),
)


def _build_index_table() -> np.ndarray:
    perm = np.asarray(_PERM, np.int64).reshape(8, 4, 4)
    perm_flat = perm.reshape(_BATCH, _STREAMS)
    b = np.arange(_BATCH)
    g = b // _GROUP_SIZE
    src_b = g[:, None] * _GROUP_SIZE + perm_flat
    s = np.arange(_STREAMS)
    src_row = src_b * _STREAMS + s[None, :]
    table = np.concatenate(
        [src_row, np.zeros((_BATCH, 4), np.int64),
         src_row, np.zeros((_BATCH, 4), np.int64)], axis=1)
    return table.astype(np.int32)


_IDX_TABLE = _build_index_table()


def _sc_body(stem3, idx_all, oh_pad, sel_out,
             idx_v, oh_v, buf0, buf1, buf2, buf3,
             sem_in0, sem_in1, sem_in2, sem_in3,
             sem_out0, sem_out1, sem_out2, sem_out3):
    wid = lax.axis_index("s") * _NC + lax.axis_index("c")
    pltpu.sync_copy(idx_all.at[wid], idx_v)
    pltpu.sync_copy(oh_pad.at[wid], oh_v)

    lanes = lax.iota(jnp.int32, 16)
    zeros16 = jnp.zeros((16,), jnp.int32)
    tabv = idx_v[...]

    v = oh_v[...]
    big = jnp.full((16,), 16, jnp.int32)
    p = jnp.min(jnp.where(v == jnp.max(v), lanes, big))

    def table_at(col):
        return jnp.max(jnp.where(lanes == col, tabv, zeros16))

    srcs = [table_at(8 + p)]
    dsts = [(sel_out, wid)]
    plan = [(r, d, c) for r, d in zip(srcs, dsts) for c in _CHUNKS]

    bufs = [buf0, buf1, buf2, buf3]
    sins = [sem_in0, sem_in1, sem_in2, sem_in3]
    souts = [sem_out0, sem_out1, sem_out2, sem_out3]

    def start_gather(j):
        k = j % _NBUF
        src, _, (c, w) = plan[j]
        return pltpu.async_copy(
            stem3.at[src, :, pl.ds(c, w)], bufs[k].at[:, pl.ds(0, w)],
            sins[k])

    def start_scatter(j):
        k = j % _NBUF
        _, (ref, row), (c, w) = plan[j]
        return pltpu.async_copy(
            bufs[k].at[:, pl.ds(0, w)], ref.at[row, :, pl.ds(c, w)],
            souts[k])

    n = len(plan)
    h_in = [None] * n
    h_out = [None] * n
    for j in range(min(_NBUF, n)):
        h_in[j] = start_gather(j)
    for j in range(n):
        h_in[j].wait()
        h_out[j] = start_scatter(j)
        if j + _NBUF < n:
            h_out[j].wait()
            h_in[j + _NBUF] = start_gather(j + _NBUF)
    for j in range(max(n - _NBUF, 0), n):
        h_out[j].wait()


@jax.jit
def _sc_call(stem3, idx_all, oh_pad):
    mesh = plsc.VectorSubcoreMesh(core_axis_name="c", subcore_axis_name="s",
                                  num_cores=_NC, num_subcores=_NS)
    return pl.kernel(
        _sc_body,
        out_type=jax.ShapeDtypeStruct((_BATCH, _CH, _T), jnp.float32),
        mesh=mesh,
        scratch_types=[
            pltpu.VMEM((16,), jnp.int32),
            pltpu.VMEM((16,), jnp.float32),
            pltpu.VMEM((_CH, _CWT), jnp.float32),
            pltpu.VMEM((_CH, _CWT), jnp.float32),
            pltpu.VMEM((_CH, _CWT), jnp.float32),
            pltpu.VMEM((_CH, _CWT), jnp.float32),
            pltpu.SemaphoreType.DMA,
            pltpu.SemaphoreType.DMA,
            pltpu.SemaphoreType.DMA,
            pltpu.SemaphoreType.DMA,
            pltpu.SemaphoreType.DMA,
            pltpu.SemaphoreType.DMA,
            pltpu.SemaphoreType.DMA,
            pltpu.SemaphoreType.DMA,
        ],
        compiler_params=pltpu.CompilerParams(needs_layout_passes=False),
    )(stem3, idx_all, oh_pad)


def _tc_stem_body(tab_ref, src, out_ref):
    del tab_ref
    out_ref[...] = src[...]


@jax.jit
def _tc_stem_call(stem3, tab128):
    grid_spec = pltpu.PrefetchScalarGridSpec(
        num_scalar_prefetch=1,
        grid=(_ROWS,),
        in_specs=[pl.BlockSpec((1, _CH, _T), lambda r, tab: (tab[r], 0, 0))],
        out_specs=pl.BlockSpec((1, _CH, _T), lambda r, tab: (r, 0, 0)),
    )
    return pl.pallas_call(
        _tc_stem_body,
        grid_spec=grid_spec,
        out_shape=jax.ShapeDtypeStruct((_ROWS, _CH, _T), jnp.float32),
    )(tab128, stem3)


def kernel(selected_stem, one_hot_vector, stem_data):
    del selected_stem
    stem3 = stem_data.reshape(_ROWS, _CH, _T)
    idx_all = jnp.asarray(_IDX_TABLE)
    tab128 = jnp.asarray(_IDX_TABLE[:, :_STREAMS].reshape(_ROWS))
    oh_pad = jnp.concatenate(
        [one_hot_vector,
         jnp.full((_BATCH, 12), -jnp.inf, jnp.float32)], axis=1)
    selected = _sc_call(stem3, idx_all, oh_pad)
    stem_out3 = _tc_stem_call(stem3, tab128)
    stem_out = stem_out3.reshape(_BATCH, _STREAMS, _CH, _T)
    return (selected, one_hot_vector, stem_out)


# speedup vs baseline: 1.8034x; 1.8034x over previous
import functools

import jax
import jax.numpy as jnp
import numpy as np
from jax import lax
from jax.experimental import pallas as pl
from jax.experimental.pallas import tpu as pltpu
from jax.experimental.pallas import tpu_sc as plsc

_GROUP_SIZE = 4
_NC = 2
_NS = 16

_BATCH, _STREAMS, _CH, _T = 32, 4, 2, 44100
_ROWS = _BATCH * _STREAMS
_CW = 11008
_CWT = _T - 3 * _CW
_CHUNKS = ((0, _CW), (_CW, _CW), (2 * _CW, _CW), (3 * _CW, _CWT))
_NBUF = 4

_PERM = (
    (1, 1, 1, 0, 0, 0, 2, 1, 2, 2, 3, 3, 3, 3, 0, 2),
    (1, 1, 1, 2, 2, 0, 0, 3, 3, 2, 2, 1, 0, 3, 3, 0),
    (0, 3, 1, 1, 2, 1, 2, 0, 3, 2, 0, 2, 1, 0, 3, 3),
    (3, 3, 0, 3, 2, 0, 2, 0, 1, 2, 1, 1, 0, 1, 3, 2),
    (0, 0, 2, 0, 3, 3, 3, 1, 1, 2, 1, 3, 2, 1, 0, 2),
    (1, 1, 1, 2, 3, 3, 3, 0, 0, 2, 2, 1, 2, 0, 0, 3),
    (3, 3, 1, 1, 0, 0, 2, 2, 1, 1, 0, 0, 2, 2, 3, 3),
    (0, 3, 2, 0, 2, 2, 0, 1, 1, 1, 3, 2, 3, 0, 1, 3),
)


def _build_index_table() -> np.ndarray:
    perm = np.asarray(_PERM, np.int64).reshape(8, 4, 4)
    perm_flat = perm.reshape(_BATCH, _STREAMS)
    b = np.arange(_BATCH)
    g = b // _GROUP_SIZE
    src_b = g[:, None] * _GROUP_SIZE + perm_flat
    s = np.arange(_STREAMS)
    src_row = src_b * _STREAMS + s[None, :]
    table = np.concatenate(
        [src_row, np.zeros((_BATCH, 4), np.int64),
         src_row, np.zeros((_BATCH, 4), np.int64)], axis=1)
    return table.astype(np.int32)


_IDX_TABLE = _build_index_table()


def _sc_body(stem3, idx_all, stem_out,
             idx_v, buf0, buf1, buf2, buf3,
             sem_in0, sem_in1, sem_in2, sem_in3,
             sem_out0, sem_out1, sem_out2, sem_out3):
    wid = lax.axis_index("s") * _NC + lax.axis_index("c")
    pltpu.sync_copy(idx_all.at[wid], idx_v)

    lanes = lax.iota(jnp.int32, 16)
    zeros16 = jnp.zeros((16,), jnp.int32)
    tabv = idx_v[...]

    def table_at(col):
        return jnp.max(jnp.where(lanes == col, tabv, zeros16))

    srcs = [table_at(j) for j in range(_STREAMS)]
    dsts = [(stem_out, wid * _STREAMS + j) for j in range(_STREAMS)]
    plan = [(r, d, c) for r, d in zip(srcs, dsts) for c in _CHUNKS]

    bufs = [buf0, buf1, buf2, buf3]
    sins = [sem_in0, sem_in1, sem_in2, sem_in3]
    souts = [sem_out0, sem_out1, sem_out2, sem_out3]

    def start_gather(j):
        k = j % _NBUF
        src, _, (c, w) = plan[j]
        return pltpu.async_copy(
            stem3.at[src, :, pl.ds(c, w)], bufs[k].at[:, pl.ds(0, w)],
            sins[k])

    def start_scatter(j):
        k = j % _NBUF
        _, (ref, row), (c, w) = plan[j]
        return pltpu.async_copy(
            bufs[k].at[:, pl.ds(0, w)], ref.at[row, :, pl.ds(c, w)],
            souts[k])

    n = len(plan)
    h_in = [None] * n
    h_out = [None] * n
    for j in range(_NBUF):
        h_in[j] = start_gather(j)
    for j in range(n):
        h_in[j].wait()
        h_out[j] = start_scatter(j)
        if j + _NBUF < n:
            h_out[j].wait()
            h_in[j + _NBUF] = start_gather(j + _NBUF)
    for j in range(n - _NBUF, n):
        h_out[j].wait()


@jax.jit
def _sc_call(stem3, idx_all):
    mesh = plsc.VectorSubcoreMesh(core_axis_name="c", subcore_axis_name="s",
                                  num_cores=_NC, num_subcores=_NS)
    return pl.kernel(
        _sc_body,
        out_type=jax.ShapeDtypeStruct((_ROWS, _CH, _T), jnp.float32),
        mesh=mesh,
        scratch_types=[
            pltpu.VMEM((16,), jnp.int32),
            pltpu.VMEM((_CH, _CWT), jnp.float32),
            pltpu.VMEM((_CH, _CWT), jnp.float32),
            pltpu.VMEM((_CH, _CWT), jnp.float32),
            pltpu.VMEM((_CH, _CWT), jnp.float32),
            pltpu.SemaphoreType.DMA,
            pltpu.SemaphoreType.DMA,
            pltpu.SemaphoreType.DMA,
            pltpu.SemaphoreType.DMA,
            pltpu.SemaphoreType.DMA,
            pltpu.SemaphoreType.DMA,
            pltpu.SemaphoreType.DMA,
            pltpu.SemaphoreType.DMA,
        ],
        compiler_params=pltpu.CompilerParams(needs_layout_passes=False),
    )(stem3, idx_all)


def _tc_sel_body(tab_ref, oh_ref, src, out_ref):
    del tab_ref, oh_ref
    out_ref[...] = src[...]


def _sel_src_map(b, tab, oh):
    w0, w1, w2, w3 = oh[b, 0], oh[b, 1], oh[b, 2], oh[b, 3]
    p = jnp.where(w1 > w0, 1, 0)
    m = jnp.maximum(w0, w1)
    p = jnp.where(w2 > m, 2, p)
    m = jnp.maximum(m, w2)
    p = jnp.where(w3 > m, 3, p)
    return tab[b, p], 0, 0


@jax.jit
def _tc_sel_call(stem3, one_hot, tab):
    grid_spec = pltpu.PrefetchScalarGridSpec(
        num_scalar_prefetch=2,
        grid=(_BATCH,),
        in_specs=[pl.BlockSpec((1, _CH, _T), _sel_src_map)],
        out_specs=pl.BlockSpec((1, _CH, _T), lambda b, tab, oh: (b, 0, 0)),
    )
    return pl.pallas_call(
        _tc_sel_body,
        grid_spec=grid_spec,
        out_shape=jax.ShapeDtypeStruct((_BATCH, _CH, _T), jnp.float32),
    )(tab, one_hot, stem3)


def kernel(selected_stem, one_hot_vector, stem_data):
    del selected_stem
    stem3 = stem_data.reshape(_ROWS, _CH, _T)
    idx_all = jnp.asarray(_IDX_TABLE)
    tab = jnp.asarray(_IDX_TABLE[:, :_STREAMS])
    stem_out3 = _sc_call(stem3, idx_all)
    selected = _tc_sel_call(stem3, one_hot_vector, tab)
    stem_out = stem_out3.reshape(_BATCH, _STREAMS, _CH, _T)
    return (selected, one_hot_vector, stem_out)
